# baseline (device time: 150374 ns/iter reference)
import jax
import jax.numpy as jnp
from jax import lax
from jax.experimental import pallas as pl
from jax.experimental.pallas import tpu as pltpu

N_DEV = 4
N_HOP = N_DEV - 1
N_SUB = 4
N_PIECE = 2 * N_SUB


def kernel(x, w_mat):
    m_per, k = x.shape
    n = w_mat.shape[1]
    half = m_per // 2
    sub = half // N_SUB
    kh = k // 2

    def body(x_hbm, w_hbm, out_hbm, x_stage, own_bf, w_stage, w_bf,
             out_stage, comm_top, comm_bot, send_t, recv_t, send_b, recv_b,
             x_sems, out_sems, w_sem):
        my = lax.axis_index("i")
        left = (my + N_DEV - 1) % N_DEV
        right = (my + 1) % N_DEV

        q_order = tuple(
            p for s in range(N_SUB) for p in (s, N_SUB + s)
        )
        x_cp = {}
        for j, q in enumerate(q_order[:2]):
            x_cp[q] = pltpu.make_async_copy(
                x_hbm.at[pl.ds(q * sub, sub), :], x_stage.at[j % 2],
                x_sems.at[j % 2],
            )
            x_cp[q].start()

        barrier_sem = pltpu.get_barrier_semaphore()
        for nbr in (left, right):
            pl.semaphore_signal(
                barrier_sem, inc=1,
                device_id=(nbr,), device_id_type=pl.DeviceIdType.MESH,
            )
        pl.semaphore_wait(barrier_sem, 2)

        def silu(v):
            return v * (1.0 / (1.0 + jnp.exp(-v)))

        def mm(a):
            return jnp.dot(a, w_bf[:, :], preferred_element_type=jnp.float32)

        started = []
        for j, q in enumerate(q_order):
            slot = j % 2
            x_cp[q].wait()
            own_bf[q, :, :] = x_stage[slot, :, :].astype(jnp.bfloat16)
            if j + 2 < N_PIECE:
                nq = q_order[j + 2]
                x_cp[nq] = pltpu.make_async_copy(
                    x_hbm.at[pl.ds(nq * sub, sub), :], x_stage.at[slot],
                    x_sems.at[slot],
                )
                x_cp[nq].start()
            if q < N_SUB:
                rdma = pltpu.make_async_remote_copy(
                    src_ref=own_bf.at[q],
                    dst_ref=comm_top.at[0, q],
                    send_sem=send_t.at[0, q], recv_sem=recv_t.at[0, q],
                    device_id=(right,), device_id_type=pl.DeviceIdType.MESH,
                )
            else:
                rdma = pltpu.make_async_remote_copy(
                    src_ref=own_bf.at[q],
                    dst_ref=comm_bot.at[0, q - N_SUB],
                    send_sem=send_b.at[0, q - N_SUB],
                    recv_sem=recv_b.at[0, q - N_SUB],
                    device_id=(left,), device_id_type=pl.DeviceIdType.MESH,
                )
            rdma.start()
            started.append(rdma)

        for i in range(2):
            cp = pltpu.make_async_copy(
                w_hbm.at[pl.ds(i * kh, kh), :], w_stage, w_sem,
            )
            cp.start()
            cp.wait()
            w_bf[pl.ds(i * kh, kh), :] = w_stage[:, :].astype(jnp.bfloat16)

        pending = [None, None]
        emit_n = [0]

        def emit(tile, row_start):
            slot = emit_n[0] % 2
            emit_n[0] += 1
            if pending[slot] is not None:
                pending[slot].wait()
            out_stage[slot, :, :] = tile
            cp = pltpu.make_async_copy(
                out_stage.at[slot],
                out_hbm.at[pl.ds(row_start, sub), :],
                out_sems.at[slot],
            )
            cp.start()
            pending[slot] = cp

        for q in range(N_PIECE):
            emit(silu(mm(own_bf[q, :, :])), my * m_per + q * sub)

        for h in range(N_HOP):
            o_t = (my + N_DEV - 1 - h) % N_DEV
            o_b = (my + 1 + h) % N_DEV
            for s in range(N_SUB):
                recv_wait_t = pltpu.make_async_remote_copy(
                    src_ref=comm_top.at[h, s], dst_ref=comm_top.at[h, s],
                    send_sem=send_t.at[h, s], recv_sem=recv_t.at[h, s],
                    device_id=(right,), device_id_type=pl.DeviceIdType.MESH,
                )
                recv_wait_t.wait_recv()
                if h + 1 < N_HOP:
                    fwd = pltpu.make_async_remote_copy(
                        src_ref=comm_top.at[h, s],
                        dst_ref=comm_top.at[h + 1, s],
                        send_sem=send_t.at[h + 1, s],
                        recv_sem=recv_t.at[h + 1, s],
                        device_id=(right,), device_id_type=pl.DeviceIdType.MESH,
                    )
                    fwd.start()
                    started.append(fwd)
                recv_wait_b = pltpu.make_async_remote_copy(
                    src_ref=comm_bot.at[h, s], dst_ref=comm_bot.at[h, s],
                    send_sem=send_b.at[h, s], recv_sem=recv_b.at[h, s],
                    device_id=(left,), device_id_type=pl.DeviceIdType.MESH,
                )
                recv_wait_b.wait_recv()
                if h + 1 < N_HOP:
                    fwd = pltpu.make_async_remote_copy(
                        src_ref=comm_bot.at[h, s],
                        dst_ref=comm_bot.at[h + 1, s],
                        send_sem=send_b.at[h + 1, s],
                        recv_sem=recv_b.at[h + 1, s],
                        device_id=(left,), device_id_type=pl.DeviceIdType.MESH,
                    )
                    fwd.start()
                    started.append(fwd)

                emit(silu(mm(comm_top[h, s, :, :])), o_t * m_per + s * sub)
                emit(silu(mm(comm_bot[h, s, :, :])),
                     o_b * m_per + half + s * sub)

        for r in started:
            r.wait_send()
        for p in pending:
            p.wait()

    return pl.pallas_call(
        body,
        out_shape=jax.ShapeDtypeStruct((N_DEV * m_per, n), jnp.float32),
        in_specs=[
            pl.BlockSpec(memory_space=pl.ANY),
            pl.BlockSpec(memory_space=pl.ANY),
        ],
        out_specs=pl.BlockSpec(memory_space=pl.ANY),
        scratch_shapes=[
            pltpu.VMEM((2, sub, k), jnp.float32),
            pltpu.VMEM((N_PIECE, sub, k), jnp.bfloat16),
            pltpu.VMEM((kh, n), jnp.float32),
            pltpu.VMEM((k, n), jnp.bfloat16),
            pltpu.VMEM((2, sub, n), jnp.float32),
            pltpu.VMEM((N_HOP, N_SUB, sub, k), jnp.bfloat16),
            pltpu.VMEM((N_HOP, N_SUB, sub, k), jnp.bfloat16),
            pltpu.SemaphoreType.DMA((N_HOP, N_SUB)),
            pltpu.SemaphoreType.DMA((N_HOP, N_SUB)),
            pltpu.SemaphoreType.DMA((N_HOP, N_SUB)),
            pltpu.SemaphoreType.DMA((N_HOP, N_SUB)),
            pltpu.SemaphoreType.DMA((2,)),
            pltpu.SemaphoreType.DMA((2,)),
            pltpu.SemaphoreType.DMA,
        ],
        compiler_params=pltpu.CompilerParams(
            collective_id=0,
            vmem_limit_bytes=100 * 1024 * 1024,
        ),
    )(x, w_mat)
